# baseline (speedup 1.0000x reference)
import jax
import jax.numpy as jnp
from jax import lax
from jax.experimental import pallas as pl
from jax.experimental.pallas import tpu as pltpu
from jax.experimental.pallas import tpu_sc as plsc

N_NODES = 10000
D_FEAT = 128
N_EDGES = 320000
EPS = 1e-6

NC = 2
NS = 16
L = 16
NW = NC * NS
E_W = N_EDGES // NW
CHUNK = 80
NCHUNK = E_W // CHUNK
NBUF = 4
NMAIN = (NCHUNK // NBUF) * NBUF
NGROUP = CHUNK // L
NSLICE = D_FEAT // L
EUNROLL = 4


def _sqrt_newton(x):
    i = plsc.bitcast(x, jnp.int32)
    i = jnp.int32(0x5F3759DF) - lax.shift_right_arithmetic(i, 1)
    y = plsc.bitcast(i, jnp.float32)
    half = x * 0.5
    for _ in range(3):
        y = y * (1.5 - half * y * y)
    return x * y


def _body(src_hbm, dst_hbm, z_hbm, out_hbm, idx_s, idx_d,
          rows_s0, rows_d0, rows_s1, rows_d1, rows_s2, rows_d2,
          rows_s3, rows_d3, tbuf, out_v,
          sem_s0, sem_d0, sem_s1, sem_d1, sem_s2, sem_d2, sem_s3, sem_d3):
    wid = lax.axis_index("s") * NC + lax.axis_index("c")
    base = wid * E_W

    pltpu.sync_copy(src_hbm.at[pl.ds(base, E_W)], idx_s)
    pltpu.sync_copy(dst_hbm.at[pl.ds(base, E_W)], idx_d)

    bufs = ((rows_s0, rows_d0, sem_s0, sem_d0),
            (rows_s1, rows_d1, sem_s1, sem_d1),
            (rows_s2, rows_d2, sem_s2, sem_d2),
            (rows_s3, rows_d3, sem_s3, sem_d3))

    def issue(j, buf):
        rs, rd, ss, sd = buf
        off = j * CHUNK
        pltpu.async_copy(z_hbm.at[idx_s.at[pl.ds(off, CHUNK)]], rs, ss)
        pltpu.async_copy(z_hbm.at[idx_d.at[pl.ds(off, CHUNK)]], rd, sd)

    def wait(buf):
        rs, rd, ss, sd = buf
        pltpu.make_async_copy(z_hbm.at[idx_s.at[pl.ds(0, CHUNK)]], rs, ss).wait()
        pltpu.make_async_copy(z_hbm.at[idx_d.at[pl.ds(0, CHUNK)]], rd, sd).wait()

    lane = lax.iota(jnp.int32, L)
    colbase = lane * L

    def compute(j, buf):
        rs, rd, _, _ = buf
        off = j * CHUNK
        for g in range(NGROUP):
            def edge_body(eq, carry2):
                for u in range(EUNROLL):
                    e = g * L + eq * EUNROLL + u
                    acc = jnp.zeros((L,), jnp.float32)
                    for s in range(NSLICE):
                        a = rs[e, pl.ds(s * L, L)]
                        b = rd[e, pl.ds(s * L, L)]
                        d = (a - b) + EPS
                        acc = acc + d * d
                    tbuf[pl.ds((eq * EUNROLL + u) * L, L)] = acc
                return carry2

            lax.fori_loop(0, L // EUNROLL, edge_body, 0)
            tot = jnp.zeros((L,), jnp.float32)
            for c in range(L):
                tot = tot + plsc.load_gather(tbuf, [colbase + c])
            out_v[pl.ds(off + g * L, L)] = _sqrt_newton(tot)

    for j in range(NBUF - 1):
        issue(j, bufs[j])

    def ring_body(p, carry):
        j0 = p * NBUF
        for b in range(NBUF):
            j = j0 + b
            jn = j + NBUF - 1
            nslot = (NBUF - 1 + b) % NBUF

            @pl.when(jn < NCHUNK)
            def _():
                issue(jn, bufs[nslot])

            wait(bufs[b])
            compute(j, bufs[b])
        return carry

    lax.fori_loop(0, NMAIN // NBUF, ring_body, 0)
    wait(bufs[NMAIN % NBUF])
    compute(NCHUNK - 1, bufs[NMAIN % NBUF])

    pltpu.sync_copy(out_v, out_hbm.at[pl.ds(base, E_W)])


def _scratch_types():
    return (
        [pltpu.VMEM((E_W,), jnp.int32)] * 2
        + [pltpu.VMEM((CHUNK, D_FEAT), jnp.float32)] * (2 * NBUF)
        + [pltpu.VMEM((L * L,), jnp.float32), pltpu.VMEM((E_W,), jnp.float32)]
        + [pltpu.SemaphoreType.DMA] * (2 * NBUF)
    )


@jax.jit
def kernel(z, edge_index):
    src = edge_index[0].astype(jnp.int32)
    dst = edge_index[1].astype(jnp.int32)
    mesh = plsc.VectorSubcoreMesh(core_axis_name="c", subcore_axis_name="s",
                                  num_cores=NC, num_subcores=NS)
    f = pl.kernel(
        _body,
        out_type=jax.ShapeDtypeStruct((N_EDGES,), jnp.float32),
        mesh=mesh,
        compiler_params=pltpu.CompilerParams(needs_layout_passes=False),
        scratch_types=_scratch_types(),
    )
    return f(src, dst, z)


# speedup vs baseline: 1.0347x; 1.0347x over previous
import jax
import jax.numpy as jnp
from jax import lax
from jax.experimental import pallas as pl
from jax.experimental.pallas import tpu as pltpu
from jax.experimental.pallas import tpu_sc as plsc

N_NODES = 10000
D_FEAT = 128
N_EDGES = 320000
EPS = 1e-6

NC = 2
NS = 16
L = 16
NW = NC * NS
E_W = N_EDGES // NW
CHUNK = 80
NCHUNK = E_W // CHUNK
NBUF = 2
NMAIN = (NCHUNK // NBUF) * NBUF
NGROUP = CHUNK // L
NSLICE = D_FEAT // L
EUNROLL = 4


def _sqrt_newton(x):
    i = plsc.bitcast(x, jnp.int32)
    i = jnp.int32(0x5F3759DF) - lax.shift_right_arithmetic(i, 1)
    y = plsc.bitcast(i, jnp.float32)
    half = x * 0.5
    for _ in range(3):
        y = y * (1.5 - half * y * y)
    return x * y


def _body(src_hbm, dst_hbm, z_hbm, out_hbm, idx_s, idx_d,
          rows_s0, rows_d0, rows_s1, rows_d1, tbuf, out_v,
          sem_s0, sem_d0, sem_s1, sem_d1):
    wid = lax.axis_index("s") * NC + lax.axis_index("c")
    base = wid * E_W

    pltpu.sync_copy(src_hbm.at[pl.ds(base, E_W)], idx_s)
    pltpu.sync_copy(dst_hbm.at[pl.ds(base, E_W)], idx_d)

    bufs = ((rows_s0, rows_d0, sem_s0, sem_d0),
            (rows_s1, rows_d1, sem_s1, sem_d1))

    def issue(j, buf):
        rs, rd, ss, sd = buf
        off = j * CHUNK
        pltpu.async_copy(z_hbm.at[idx_s.at[pl.ds(off, CHUNK)]], rs, ss)
        pltpu.async_copy(z_hbm.at[idx_d.at[pl.ds(off, CHUNK)]], rd, sd)

    def wait(buf):
        rs, rd, ss, sd = buf
        pltpu.make_async_copy(z_hbm.at[idx_s.at[pl.ds(0, CHUNK)]], rs, ss).wait()
        pltpu.make_async_copy(z_hbm.at[idx_d.at[pl.ds(0, CHUNK)]], rd, sd).wait()

    lane = lax.iota(jnp.int32, L)
    colbase = lane * L

    def compute(j, buf):
        rs, rd, _, _ = buf
        off = j * CHUNK
        for g in range(NGROUP):
            def edge_body(eq, carry2):
                for u in range(EUNROLL):
                    e = g * L + eq * EUNROLL + u
                    acc = jnp.zeros((L,), jnp.float32)
                    for s in range(NSLICE):
                        a = rs[e, pl.ds(s * L, L)]
                        b = rd[e, pl.ds(s * L, L)]
                        d = (a - b) + EPS
                        acc = acc + d * d
                    tbuf[pl.ds((eq * EUNROLL + u) * L, L)] = acc
                return carry2

            lax.fori_loop(0, L // EUNROLL, edge_body, 0)
            tot = jnp.zeros((L,), jnp.float32)
            for c in range(L):
                tot = tot + plsc.load_gather(tbuf, [colbase + c])
            out_v[pl.ds(off + g * L, L)] = _sqrt_newton(tot)

    for j in range(NBUF - 1):
        issue(j, bufs[j])

    def ring_body(p, carry):
        j0 = p * NBUF
        for b in range(NBUF):
            j = j0 + b
            jn = j + NBUF - 1
            nslot = (NBUF - 1 + b) % NBUF

            @pl.when(jn < NCHUNK)
            def _():
                issue(jn, bufs[nslot])

            wait(bufs[b])
            compute(j, bufs[b])
        return carry

    lax.fori_loop(0, NMAIN // NBUF, ring_body, 0)
    wait(bufs[NMAIN % NBUF])
    compute(NCHUNK - 1, bufs[NMAIN % NBUF])

    pltpu.sync_copy(out_v, out_hbm.at[pl.ds(base, E_W)])


def _scratch_types():
    return (
        [pltpu.VMEM((E_W,), jnp.int32)] * 2
        + [pltpu.VMEM((CHUNK, D_FEAT), jnp.float32)] * (2 * NBUF)
        + [pltpu.VMEM((L * L,), jnp.float32), pltpu.VMEM((E_W,), jnp.float32)]
        + [pltpu.SemaphoreType.DMA] * (2 * NBUF)
    )


@jax.jit
def kernel(z, edge_index):
    src = edge_index[0].astype(jnp.int32)
    dst = edge_index[1].astype(jnp.int32)
    mesh = plsc.VectorSubcoreMesh(core_axis_name="c", subcore_axis_name="s",
                                  num_cores=NC, num_subcores=NS)
    f = pl.kernel(
        _body,
        out_type=jax.ShapeDtypeStruct((N_EDGES,), jnp.float32),
        mesh=mesh,
        compiler_params=pltpu.CompilerParams(needs_layout_passes=False),
        scratch_types=_scratch_types(),
    )
    return f(src, dst, z)


# speedup vs baseline: 1.1620x; 1.1231x over previous
import jax
import jax.numpy as jnp
from jax import lax
from jax.experimental import pallas as pl
from jax.experimental.pallas import tpu as pltpu
from jax.experimental.pallas import tpu_sc as plsc

N_NODES = 10000
D_FEAT = 128
N_EDGES = 320000
EPS = 1e-6

NC = 2
NS = 16
L = 16
NW = NC * NS
E_W = N_EDGES // NW
CHUNK = 80
NCHUNK = E_W // CHUNK
NGROUP = CHUNK // L
NSLICE = D_FEAT // L
OUT_CHUNKS = 64
NPAIR1 = OUT_CHUNKS // 2
NPAIR2 = (NCHUNK - 1 - OUT_CHUNKS) // 2


def _sqrt_newton(x):
    i = plsc.bitcast(x, jnp.int32)
    i = jnp.int32(0x5F3759DF) - lax.shift_right_arithmetic(i, 1)
    y = plsc.bitcast(i, jnp.float32)
    half = x * 0.5
    for _ in range(3):
        y = y * (1.5 - half * y * y)
    return x * y


def _body(src_hbm, dst_hbm, z_hbm, out_hbm,
          idx_s0, idx_d0, idx_s1, idx_d1,
          rows_s0, rows_d0, rows_s1, rows_d1, tbuf, out_v, shared,
          sem_is0, sem_id0, sem_is1, sem_id1,
          sem_s0, sem_d0, sem_s1, sem_d1):
    sid = lax.axis_index("s")
    wid = sid * NC + lax.axis_index("c")
    base = wid * E_W

    @pl.when(sid == 0)
    def _():
        pltpu.sync_copy(z_hbm, shared)
    plsc.subcore_barrier()

    ibufs = ((idx_s0, idx_d0, sem_is0, sem_id0),
             (idx_s1, idx_d1, sem_is1, sem_id1))
    rbufs = ((rows_s0, rows_d0, sem_s0, sem_d0),
             (rows_s1, rows_d1, sem_s1, sem_d1))

    def idx_fetch(j, ib):
        xs, xd, ss, sd = ib
        off = base + j * CHUNK
        pltpu.async_copy(src_hbm.at[pl.ds(off, CHUNK)], xs, ss)
        pltpu.async_copy(dst_hbm.at[pl.ds(off, CHUNK)], xd, sd)

    def idx_wait(ib):
        xs, xd, ss, sd = ib
        pltpu.make_async_copy(src_hbm.at[pl.ds(0, CHUNK)], xs, ss).wait()
        pltpu.make_async_copy(dst_hbm.at[pl.ds(0, CHUNK)], xd, sd).wait()

    def issue(ib, rb):
        xs, xd, _, _ = ib
        rs, rd, ss, sd = rb
        pltpu.async_copy(shared.at[xs], rs, ss)
        pltpu.async_copy(shared.at[xd], rd, sd)

    def wait(ib, rb):
        xs, xd, _, _ = ib
        rs, rd, ss, sd = rb
        pltpu.make_async_copy(shared.at[xs], rs, ss).wait()
        pltpu.make_async_copy(shared.at[xd], rd, sd).wait()

    lane = lax.iota(jnp.int32, L)
    colbase = lane * L

    def compute(out_off, rb):
        rs, rd, _, _ = rb
        for g in range(NGROUP):
            def edge_body(el, carry2):
                e = g * L + el
                acc = jnp.zeros((L,), jnp.float32)
                for s in range(NSLICE):
                    a = rs[e, pl.ds(s * L, L)]
                    b = rd[e, pl.ds(s * L, L)]
                    d = (a - b) + EPS
                    acc = acc + d * d
                tbuf[pl.ds(el * L, L)] = acc
                return carry2

            lax.fori_loop(0, L, edge_body, 0)
            tot = jnp.zeros((L,), jnp.float32)
            for c in range(L):
                tot = tot + plsc.load_gather(tbuf, [colbase + c])
            out_v[pl.ds(out_off + g * L, L)] = _sqrt_newton(tot)

    pltpu.sync_copy(src_hbm.at[pl.ds(base, CHUNK)], idx_s0)
    pltpu.sync_copy(dst_hbm.at[pl.ds(base, CHUNK)], idx_d0)
    issue(ibufs[0], rbufs[0])
    idx_fetch(1, ibufs[1])

    def pair_body(p, phase_base):
        j0 = 2 * p
        idx_wait(ibufs[1])
        issue(ibufs[1], rbufs[1])
        wait(ibufs[0], rbufs[0])
        idx_fetch(j0 + 2, ibufs[0])
        compute((j0 - phase_base) * CHUNK, rbufs[0])
        idx_wait(ibufs[0])
        issue(ibufs[0], rbufs[0])
        wait(ibufs[1], rbufs[1])

        @pl.when(j0 + 3 < NCHUNK)
        def _():
            idx_fetch(j0 + 3, ibufs[1])

        compute((j0 + 1 - phase_base) * CHUNK, rbufs[1])
        return phase_base

    lax.fori_loop(0, NPAIR1, pair_body, 0)
    pltpu.sync_copy(out_v.at[pl.ds(0, OUT_CHUNKS * CHUNK)],
                    out_hbm.at[pl.ds(base, OUT_CHUNKS * CHUNK)])

    lax.fori_loop(NPAIR1, NPAIR1 + NPAIR2, pair_body, OUT_CHUNKS)
    wait(ibufs[0], rbufs[0])
    compute((NCHUNK - 1 - OUT_CHUNKS) * CHUNK, rbufs[0])
    rest = (NCHUNK - OUT_CHUNKS) * CHUNK
    pltpu.sync_copy(out_v.at[pl.ds(0, rest)],
                    out_hbm.at[pl.ds(base + OUT_CHUNKS * CHUNK, rest)])


def _scratch_types():
    return (
        [pltpu.VMEM((CHUNK,), jnp.int32)] * 4
        + [pltpu.VMEM((CHUNK, D_FEAT), jnp.float32)] * 4
        + [pltpu.VMEM((L * L,), jnp.float32),
           pltpu.VMEM((OUT_CHUNKS * CHUNK,), jnp.float32)]
        + [pltpu.VMEM_SHARED((N_NODES, D_FEAT), jnp.float32)]
        + [pltpu.SemaphoreType.DMA] * 8
    )


@jax.jit
def kernel(z, edge_index):
    src = edge_index[0].astype(jnp.int32)
    dst = edge_index[1].astype(jnp.int32)
    mesh = plsc.VectorSubcoreMesh(core_axis_name="c", subcore_axis_name="s",
                                  num_cores=NC, num_subcores=NS)
    f = pl.kernel(
        _body,
        out_type=jax.ShapeDtypeStruct((N_EDGES,), jnp.float32),
        mesh=mesh,
        compiler_params=pltpu.CompilerParams(needs_layout_passes=False),
        scratch_types=_scratch_types(),
    )
    return f(src, dst, z)
